# baseline (speedup 1.0000x reference)
import jax
import jax.numpy as jnp
from jax.experimental import pallas as pl


def _router_block(x_ref, w_ref, b_ref, o_ref):
    logits = jnp.dot(x_ref[...], w_ref[...], preferred_element_type=jnp.float32)
    logits = logits + b_ref[...]
    m = jnp.max(logits, axis=-1, keepdims=True)
    e = jnp.exp(logits - m)
    o_ref[...] = e / jnp.sum(e, axis=-1, keepdims=True)


def kernel(x, W, b):
    N, D = x.shape
    E = W.shape[0]
    BLOCK = 2048
    Wt = W.T
    b2 = b.reshape(1, E)
    out = pl.pallas_call(
        _router_block,
        grid=(N // BLOCK,),
        in_specs=[
            pl.BlockSpec((BLOCK, D), lambda i: (i, 0)),
            pl.BlockSpec((D, E), lambda i: (0, 0)),
            pl.BlockSpec((1, E), lambda i: (0, 0)),
        ],
        out_specs=pl.BlockSpec((BLOCK, E), lambda i: (i, 0)),
        out_shape=jax.ShapeDtypeStruct((N, E), jnp.float32),
    )(x, Wt, b2)
    return out
